# Initial kernel scaffold; baseline (speedup 1.0000x reference)
import jax
import jax.numpy as jnp
from jax.experimental import pallas as pl


def kernel(msa_seq, mask, query_seq, msa_table, pos_table, W, b, gamma, beta):
    raise NotImplementedError("write your pallas kernel here")


# speedup vs baseline: 5.7272x; 5.7272x over previous
import functools

import jax
import jax.numpy as jnp
from jax import lax
from jax.experimental import pallas as pl
from jax.experimental.pallas import tpu as pltpu

B, N, L, D, V = 2, 128, 1024, 256, 21
VP = 32
NBLK = 8
LBLK = 512


def _body(seq_ref, mask_ref, qseq_ref, tab_ref, pos_ref, wt_ref, bias_ref,
          g_ref, beta_ref, out_ref, base_ref):
    nb = pl.program_id(2)

    @pl.when(nb == 0)
    def _compute_base():
        qtok = qseq_ref[0, 0, :]
        oh_q = (qtok[:, None]
                == lax.broadcasted_iota(jnp.int32, (LBLK, VP), 1)
                ).astype(jnp.float32)
        qe = jnp.dot(oh_q, tab_ref[...], precision=lax.Precision.HIGHEST)
        q = jnp.dot(qe, wt_ref[...], precision=lax.Precision.HIGHEST)
        base_ref[...] = pos_ref[...] + q + bias_ref[0, :]

    base = base_ref[...]
    for n in range(NBLK):
        tok = seq_ref[0, n, :]
        oh = (tok[:, None]
              == lax.broadcasted_iota(jnp.int32, (LBLK, VP), 1)
              ).astype(jnp.float32)
        emb = jnp.dot(oh, tab_ref[...], precision=lax.Precision.HIGHEST)
        x = (emb + base) * mask_ref[0, n, :][:, None]
        mu = jnp.mean(x, axis=-1, keepdims=True)
        xc = x - mu
        var = jnp.mean(xc * xc, axis=-1, keepdims=True)
        y = xc * lax.rsqrt(var + 1e-5) * g_ref[0, :] + beta_ref[0, :]
        out_ref[0, n] = y


@jax.jit
def kernel(msa_seq, mask, query_seq, msa_table, pos_table, W, b, gamma, beta):
    tab = jnp.zeros((VP, D), jnp.float32).at[:V].set(msa_table)
    wt = W.T
    grid = (B, L // LBLK, N // NBLK)
    return pl.pallas_call(
        _body,
        grid=grid,
        in_specs=[
            pl.BlockSpec((1, NBLK, LBLK), lambda bi, lb, nb: (bi, nb, lb)),
            pl.BlockSpec((1, NBLK, LBLK), lambda bi, lb, nb: (bi, nb, lb)),
            pl.BlockSpec((1, 1, LBLK), lambda bi, lb, nb: (bi, 0, lb)),
            pl.BlockSpec((VP, D), lambda bi, lb, nb: (0, 0)),
            pl.BlockSpec((LBLK, D), lambda bi, lb, nb: (lb, 0)),
            pl.BlockSpec((D, D), lambda bi, lb, nb: (0, 0)),
            pl.BlockSpec((1, D), lambda bi, lb, nb: (0, 0)),
            pl.BlockSpec((1, D), lambda bi, lb, nb: (0, 0)),
            pl.BlockSpec((1, D), lambda bi, lb, nb: (0, 0)),
        ],
        out_specs=pl.BlockSpec((1, NBLK, LBLK, D),
                               lambda bi, lb, nb: (bi, nb, lb, 0)),
        out_shape=jax.ShapeDtypeStruct((B, N, L, D), jnp.float32),
        scratch_shapes=[pltpu.VMEM((LBLK, D), jnp.float32)],
        compiler_params=pltpu.CompilerParams(
            dimension_semantics=("arbitrary", "arbitrary", "arbitrary"),
        ),
    )(msa_seq, mask, query_seq.reshape(B, 1, L), tab, pos_table, wt,
      b.reshape(1, D), gamma.reshape(1, D), beta.reshape(1, D))
